# baseline (speedup 1.0000x reference)
import functools

import jax
import jax.numpy as jnp
from jax import lax
from jax.experimental import pallas as pl
from jax.experimental.pallas import tpu as pltpu
from jax.experimental.pallas import tpu_sc as plsc

N = 8192
D = 256
KNN = 16


BR1 = 256
SR = 8
G1 = N // BR1

_HIGH = jax.lax.Precision.HIGHEST


def _bdot(a, b):
    return jax.lax.dot_general(
        a.astype(jnp.bfloat16), b.astype(jnp.bfloat16),
        (((1,), (1,)), ((), ())), preferred_element_type=jnp.float32)


def _k1_body(z_ref, xxt_ref, xxc_ref, w1_ref, b1_ref, idx_ref, h1_ref,
             xy_ref):
    i = pl.program_id(0)

    zb = z_ref[pl.ds(i * BR1, BR1), :]
    zball = z_ref[...]
    xy_ref[...] = _bdot(zb, zball)
    h1_ref[...] = jax.nn.relu(
        jax.lax.dot_general(zb.astype(jnp.bfloat16),
                            w1_ref[...].astype(jnp.bfloat16),
                            (((1,), (0,)), ((), ())),
                            preferred_element_type=jnp.float32) + b1_ref[...])
    xxt = xxt_ref[...]
    big = jnp.int32(1 << 30)
    BIG = jnp.float32(1e30)
    NCH = N // 128
    colio = jax.lax.broadcasted_iota(jnp.int32, (SR, N), 1)
    cols64 = jax.lax.broadcasted_iota(jnp.int32, (SR, NCH), 1)
    QR = SR * NCH
    PR = SR * KNN
    r_p = jax.lax.broadcasted_iota(jnp.int32, (PR, QR), 0) // KNN
    r_q = jax.lax.broadcasted_iota(jnp.int32, (PR, QR), 1) // NCH
    c_q = jax.lax.broadcasted_iota(jnp.int32, (PR, QR), 1) % NCH
    lane128 = jax.lax.broadcasted_iota(jnp.int32, (PR, 128), 1)
    CW = KNN * 128
    e_rows = (jax.lax.broadcasted_iota(jnp.int32, (PR, SR), 0) // KNN
              == jax.lax.broadcasted_iota(jnp.int32, (PR, SR), 1)
              ).astype(jnp.float32)
    s_p = jax.lax.broadcasted_iota(jnp.int32, (PR, KNN), 0) % KNN
    slot_io = jax.lax.broadcasted_iota(jnp.int32, (PR, KNN), 1)

    def _sub(j, _):
        xy = xy_ref[pl.ds(j * SR, SR), :]
        xxb = xxc_ref[pl.ds(j * SR, SR), :]
        dist2 = (xxb + xxt) - 2.0 * xy
        dist = jnp.sqrt(jnp.maximum(dist2, 0.0) + 1e-8)
        rowio = (jax.lax.broadcasted_iota(jnp.int32, (SR, N), 0)
                 + i * BR1 + j * SR)
        dist = jnp.where(rowio == colio, BIG, dist)

        dr = dist.reshape(QR, 128)
        cm = jnp.min(dr, axis=1, keepdims=True).reshape(SR, NCH)
        mcur = cm
        sel = []
        for _ in range(KNN):
            mm = jnp.min(mcur, axis=1, keepdims=True)
            kk = jnp.where(mcur == mm, cols64, big)
            cc = jnp.min(kk, axis=1, keepdims=True)
            sel.append(cc)
            mcur = jnp.where(kk == cc, BIG, mcur)
        scol = jnp.concatenate(sel, axis=1)
        scol_sel = jax.lax.dot_general(
            e_rows, scol.astype(jnp.float32), (((1,), (0,)), ((), ())),
            precision=jax.lax.Precision.HIGHEST)
        scol_r = jnp.sum(jnp.where(slot_io == s_p, scol_sel, 0.0),
                         axis=1, keepdims=True).astype(jnp.int32)
        s_mat = ((r_p == r_q) & (scol_r == c_q)).astype(jnp.float32)
        comp = jax.lax.dot_general(
            s_mat, dr, (((1,), (0,)), ((), ())),
            precision=jax.lax.Precision.HIGHEST)
        gcol = scol_r * 128 + lane128

        dcur = comp.reshape(SR, CW)
        gc = gcol.reshape(SR, CW)
        out = []
        for _ in range(KNN):
            m = jnp.min(dcur, axis=1, keepdims=True)
            key = jnp.where(dcur == m, gc, big)
            c = jnp.min(key, axis=1, keepdims=True)
            out.append(c)
            dcur = jnp.where(key == c, BIG, dcur)
        idx_ref[pl.ds(j * SR, SR), :] = jnp.concatenate(out, axis=1)
        return 0

    lax.fori_loop(0, BR1 // SR, _sub, 0)


def _run_k1(z, xxt, xxc, W1, b1):
    return pl.pallas_call(
        _k1_body,
        grid=(G1,),
        in_specs=[
            pl.BlockSpec((N, D), lambda i: (0, 0)),
            pl.BlockSpec((1, N), lambda i: (0, 0)),
            pl.BlockSpec((BR1, 1), lambda i: (i, 0)),
            pl.BlockSpec((D, D), lambda i: (0, 0)),
            pl.BlockSpec((1, D), lambda i: (0, 0)),
        ],
        out_specs=[
            pl.BlockSpec((BR1, KNN), lambda i: (i, 0)),
            pl.BlockSpec((BR1, D), lambda i: (i, 0)),
        ],
        out_shape=[
            jax.ShapeDtypeStruct((N, KNN), jnp.int32),
            jax.ShapeDtypeStruct((N, D), jnp.float32),
        ],
        scratch_shapes=[
            pltpu.VMEM((BR1, N), jnp.float32),
        ],
    )(z, xxt, xxc, W1, b1)


NC = 2
NS = 16
HALF = N // NC
RF = HALF // NS
RV = N // NS
ZCH = 32768
ZITER = HALF * N // NS // ZCH


def _k2_body(idx_hbm, a_hbm, idx_f, idx_r, zbuf, ones, sem):
    c = lax.axis_index("c")
    s = lax.axis_index("s")

    def _zinit(t, _):
        zbuf[pl.ds(t * 16, 16)] = jnp.zeros((16,), jnp.float32)
        return 0
    lax.fori_loop(0, ZCH // 16, _zinit, 0)
    ones[...] = jnp.ones((16,), jnp.float32)

    tile_base = (c * HALF + s * RF) * N

    def _zfill(t, _):
        pltpu.sync_copy(zbuf, a_hbm.at[pl.ds(tile_base + t * ZCH, ZCH)])
        return 0
    lax.fori_loop(0, ZITER, _zfill, 0)

    plsc.subcore_barrier()

    rf_base = c * HALF + s * RF
    pltpu.sync_copy(idx_hbm.at[pl.ds(rf_base * KNN, RF * KNN)], idx_f)
    rr_base = s * RV
    pltpu.sync_copy(idx_hbm.at[pl.ds(rr_base * KNN, RV * KNN)], idx_r)

    safe = rf_base * N + idx_f[pl.ds(0, KNN)]

    def _fwd(g, _):
        ds = []
        for u in range(8):
            r = g * 8 + u
            p = (rf_base + r) * N + idx_f[pl.ds(r * KNN, KNN)]
            ds.append(pltpu.async_copy(ones, a_hbm.at[p], sem))
        for d in ds:
            d.wait()
        return 0
    lax.fori_loop(0, RF // 8, _fwd, 0)

    lo = c * HALF
    hi = lo + HALF

    def _rev(g, _):
        ds = []
        for u in range(8):
            r = g * 8 + u
            v = idx_r[pl.ds(r * KNN, KNN)]
            p = v * N + (rr_base + r)
            keep = (v >= lo) & (v < hi)
            p = jnp.where(keep, p, safe)
            ds.append(pltpu.async_copy(ones, a_hbm.at[p], sem))
        for d in ds:
            d.wait()
        return 0
    lax.fori_loop(0, RV // 8, _rev, 0)


def _run_k2(idx):
    mesh = plsc.VectorSubcoreMesh(core_axis_name="c", subcore_axis_name="s")
    f = functools.partial(
        pl.kernel,
        out_type=jax.ShapeDtypeStruct((N * N,), jnp.float32),
        mesh=mesh,
        scratch_types=[
            pltpu.VMEM((RF * KNN,), jnp.int32),
            pltpu.VMEM((RV * KNN,), jnp.int32),
            pltpu.VMEM((ZCH,), jnp.float32),
            pltpu.VMEM((16,), jnp.float32),
            pltpu.SemaphoreType.DMA,
        ],
    )(_k2_body)
    return f(idx.reshape(N * KNN))


BR3 = 256
CK = 1024
KS = N // CK


def _k3_body(a_ref, h1_ref, w2_ref, b2_ref, out_ref, acc_ref, deg_ref):
    k = pl.program_id(1)

    @pl.when(k == 0)
    def _():
        acc_ref[...] = jnp.zeros_like(acc_ref)
        deg_ref[...] = jnp.zeros_like(deg_ref)

    a = a_ref[...]
    hb = h1_ref[pl.ds(k * CK, CK), :]
    acc_ref[...] += jax.lax.dot_general(
        a.astype(jnp.bfloat16), hb.astype(jnp.bfloat16),
        (((1,), (0,)), ((), ())), preferred_element_type=jnp.float32)
    deg_ref[...] += jnp.sum(a, axis=1, keepdims=True)

    @pl.when(k == KS - 1)
    def _():
        agg = acc_ref[...] / (deg_ref[...] + 1e-6)
        out_ref[...] = jax.nn.relu(
            jax.lax.dot_general(
                agg.astype(jnp.bfloat16), w2_ref[...].astype(jnp.bfloat16),
                (((1,), (0,)), ((), ())),
                preferred_element_type=jnp.float32) + b2_ref[...])


def _run_k3(A, h1, W2, b2):
    return pl.pallas_call(
        _k3_body,
        grid=(N // BR3, KS),
        in_specs=[
            pl.BlockSpec((BR3, CK), lambda i, k: (i, k)),
            pl.BlockSpec((N, D), lambda i, k: (0, 0)),
            pl.BlockSpec((D, D), lambda i, k: (0, 0)),
            pl.BlockSpec((1, D), lambda i, k: (0, 0)),
        ],
        out_specs=pl.BlockSpec((BR3, D), lambda i, k: (i, 0)),
        out_shape=jax.ShapeDtypeStruct((N, D), jnp.float32),
        scratch_shapes=[
            pltpu.VMEM((BR3, D), jnp.float32),
            pltpu.VMEM((BR3, 1), jnp.float32),
        ],
        compiler_params=pltpu.CompilerParams(
            dimension_semantics=("parallel", "arbitrary")),
    )(A, h1, W2, b2)


def kernel(z, W1, b1, W2, b2):
    xxc = jnp.sum(z ** 2, axis=1, keepdims=True)
    xxt = xxc.reshape(1, N)
    idx, h1 = _run_k1(z, xxt, xxc, W1, b1.reshape(1, D))
    a_flat = _run_k2(idx)
    A = a_flat.reshape(N, N)
    h = _run_k3(A, h1, W2, b2.reshape(1, D))
    return (h, A)


# speedup vs baseline: 1.6634x; 1.6634x over previous
import functools

import jax
import jax.numpy as jnp
from jax import lax
from jax.experimental import pallas as pl
from jax.experimental.pallas import tpu as pltpu
from jax.experimental.pallas import tpu_sc as plsc

N = 8192
D = 256
KNN = 16


BR1 = 256
SR = 8
G1 = N // BR1

_HIGH = jax.lax.Precision.HIGHEST


def _bdot(a, b):
    return jax.lax.dot_general(
        a.astype(jnp.bfloat16), b.astype(jnp.bfloat16),
        (((1,), (1,)), ((), ())), preferred_element_type=jnp.float32)


def _k1_body(z_ref, xxt_ref, xxc_ref, w1_ref, b1_ref, idx_ref, h1_ref,
             xy_ref):
    i = pl.program_id(0)

    zb = z_ref[pl.ds(i * BR1, BR1), :]
    zball = z_ref[...]
    xy_ref[...] = _bdot(zb, zball)
    h1_ref[...] = jax.nn.relu(
        jax.lax.dot_general(zb.astype(jnp.bfloat16),
                            w1_ref[...].astype(jnp.bfloat16),
                            (((1,), (0,)), ((), ())),
                            preferred_element_type=jnp.float32) + b1_ref[...])
    xxt = xxt_ref[...]
    big = jnp.int32(1 << 30)
    colio = jax.lax.broadcasted_iota(jnp.int32, (SR, N), 1)
    BLK = 1024
    NB = N // BLK
    cio_b = jax.lax.broadcasted_iota(jnp.int32, (SR, BLK), 1)

    def _sub(j, _):
        xy = xy_ref[pl.ds(j * SR, SR), :]
        xxb = xxc_ref[pl.ds(j * SR, SR), :]
        dist2 = (xxb + xxt) - 2.0 * xy
        dist = jnp.sqrt(jnp.maximum(dist2, 0.0) + 1e-8)
        rowio = (jax.lax.broadcasted_iota(jnp.int32, (SR, N), 0)
                 + i * BR1 + j * SR)
        dall = jnp.where(rowio == colio, jnp.inf, dist)
        cand_v = []
        cand_i = []
        for b in range(NB):
            db = jax.lax.slice(dall, (0, b * BLK), (SR, (b + 1) * BLK))
            cio = cio_b + b * BLK
            m = jnp.min(db, axis=1, keepdims=True)
            for t in range(KNN):
                key = jnp.where(db == m, cio, big)
                c = jnp.min(key, axis=1, keepdims=True)
                cand_v.append(m)
                cand_i.append(c)
                db = jnp.where(key == c, jnp.inf, db)
                m = jnp.min(db, axis=1, keepdims=True)
        vv = jnp.concatenate(cand_v, axis=1)
        ii = jnp.concatenate(cand_i, axis=1)
        out = []
        for t in range(KNN):
            m = jnp.min(vv, axis=1, keepdims=True)
            key = jnp.where(vv == m, ii, big)
            c = jnp.min(key, axis=1, keepdims=True)
            out.append(c)
            vv = jnp.where(key == c, jnp.inf, vv)
        idx_ref[pl.ds(j * SR, SR), :] = jnp.concatenate(out, axis=1)
        return 0

    lax.fori_loop(0, BR1 // SR, _sub, 0)


def _run_k1(z, xxt, xxc, W1, b1):
    return pl.pallas_call(
        _k1_body,
        grid=(G1,),
        in_specs=[
            pl.BlockSpec((N, D), lambda i: (0, 0)),
            pl.BlockSpec((1, N), lambda i: (0, 0)),
            pl.BlockSpec((BR1, 1), lambda i: (i, 0)),
            pl.BlockSpec((D, D), lambda i: (0, 0)),
            pl.BlockSpec((1, D), lambda i: (0, 0)),
        ],
        out_specs=[
            pl.BlockSpec((BR1, KNN), lambda i: (i, 0)),
            pl.BlockSpec((BR1, D), lambda i: (i, 0)),
        ],
        out_shape=[
            jax.ShapeDtypeStruct((N, KNN), jnp.int32),
            jax.ShapeDtypeStruct((N, D), jnp.float32),
        ],
        scratch_shapes=[
            pltpu.VMEM((BR1, N), jnp.float32),
        ],
    )(z, xxt, xxc, W1, b1)


NC = 2
NS = 16
HALF = N // NC
RF = HALF // NS
RV = N // NS
ZCH = 32768
ZITER = HALF * N // NS // ZCH


def _k2_body(idx_hbm, a_hbm, idx_f, idx_r, zbuf, ones, sem):
    c = lax.axis_index("c")
    s = lax.axis_index("s")

    def _zinit(t, _):
        zbuf[pl.ds(t * 16, 16)] = jnp.zeros((16,), jnp.float32)
        return 0
    lax.fori_loop(0, ZCH // 16, _zinit, 0)
    ones[...] = jnp.ones((16,), jnp.float32)

    tile_base = (c * HALF + s * RF) * N

    def _zfill(t, _):
        pltpu.sync_copy(zbuf, a_hbm.at[pl.ds(tile_base + t * ZCH, ZCH)])
        return 0
    lax.fori_loop(0, ZITER, _zfill, 0)

    plsc.subcore_barrier()

    rf_base = c * HALF + s * RF
    pltpu.sync_copy(idx_hbm.at[pl.ds(rf_base * KNN, RF * KNN)], idx_f)
    rr_base = s * RV
    pltpu.sync_copy(idx_hbm.at[pl.ds(rr_base * KNN, RV * KNN)], idx_r)

    safe = rf_base * N + idx_f[pl.ds(0, KNN)]

    def _fwd(g, _):
        ds = []
        for u in range(8):
            r = g * 8 + u
            p = (rf_base + r) * N + idx_f[pl.ds(r * KNN, KNN)]
            ds.append(pltpu.async_copy(ones, a_hbm.at[p], sem))
        for d in ds:
            d.wait()
        return 0
    lax.fori_loop(0, RF // 8, _fwd, 0)

    lo = c * HALF
    hi = lo + HALF

    def _rev(g, _):
        ds = []
        for u in range(8):
            r = g * 8 + u
            v = idx_r[pl.ds(r * KNN, KNN)]
            p = v * N + (rr_base + r)
            keep = (v >= lo) & (v < hi)
            p = jnp.where(keep, p, safe)
            ds.append(pltpu.async_copy(ones, a_hbm.at[p], sem))
        for d in ds:
            d.wait()
        return 0
    lax.fori_loop(0, RV // 8, _rev, 0)


def _run_k2(idx):
    mesh = plsc.VectorSubcoreMesh(core_axis_name="c", subcore_axis_name="s")
    f = functools.partial(
        pl.kernel,
        out_type=jax.ShapeDtypeStruct((N * N,), jnp.float32),
        mesh=mesh,
        scratch_types=[
            pltpu.VMEM((RF * KNN,), jnp.int32),
            pltpu.VMEM((RV * KNN,), jnp.int32),
            pltpu.VMEM((ZCH,), jnp.float32),
            pltpu.VMEM((16,), jnp.float32),
            pltpu.SemaphoreType.DMA,
        ],
    )(_k2_body)
    return f(idx.reshape(N * KNN))


BR3 = 256
CK = 1024
KS = N // CK


def _k3_body(a_ref, h1_ref, w2_ref, b2_ref, out_ref, acc_ref, deg_ref):
    k = pl.program_id(1)

    @pl.when(k == 0)
    def _():
        acc_ref[...] = jnp.zeros_like(acc_ref)
        deg_ref[...] = jnp.zeros_like(deg_ref)

    a = a_ref[...]
    hb = h1_ref[pl.ds(k * CK, CK), :]
    acc_ref[...] += jax.lax.dot_general(
        a.astype(jnp.bfloat16), hb.astype(jnp.bfloat16),
        (((1,), (0,)), ((), ())), preferred_element_type=jnp.float32)
    deg_ref[...] += jnp.sum(a, axis=1, keepdims=True)

    @pl.when(k == KS - 1)
    def _():
        agg = acc_ref[...] / (deg_ref[...] + 1e-6)
        out_ref[...] = jax.nn.relu(
            jax.lax.dot_general(
                agg.astype(jnp.bfloat16), w2_ref[...].astype(jnp.bfloat16),
                (((1,), (0,)), ((), ())),
                preferred_element_type=jnp.float32) + b2_ref[...])


def _run_k3(A, h1, W2, b2):
    return pl.pallas_call(
        _k3_body,
        grid=(N // BR3, KS),
        in_specs=[
            pl.BlockSpec((BR3, CK), lambda i, k: (i, k)),
            pl.BlockSpec((N, D), lambda i, k: (0, 0)),
            pl.BlockSpec((D, D), lambda i, k: (0, 0)),
            pl.BlockSpec((1, D), lambda i, k: (0, 0)),
        ],
        out_specs=pl.BlockSpec((BR3, D), lambda i, k: (i, 0)),
        out_shape=jax.ShapeDtypeStruct((N, D), jnp.float32),
        scratch_shapes=[
            pltpu.VMEM((BR3, D), jnp.float32),
            pltpu.VMEM((BR3, 1), jnp.float32),
        ],
        compiler_params=pltpu.CompilerParams(
            dimension_semantics=("parallel", "arbitrary")),
    )(A, h1, W2, b2)


def kernel(z, W1, b1, W2, b2):
    xxc = jnp.sum(z ** 2, axis=1, keepdims=True)
    xxt = xxc.reshape(1, N)
    idx, h1 = _run_k1(z, xxt, xxc, W1, b1.reshape(1, D))
    a_flat = _run_k2(idx)
    A = a_flat.reshape(N, N)
    h = _run_k3(A, h1, W2, b2.reshape(1, D))
    return (h, A)


# speedup vs baseline: 2.5667x; 1.5430x over previous
import functools

import jax
import jax.numpy as jnp
from jax import lax
from jax.experimental import pallas as pl
from jax.experimental.pallas import tpu as pltpu
from jax.experimental.pallas import tpu_sc as plsc

N = 8192
D = 256
KNN = 16


BR1 = 256
SR = 8
G1 = N // BR1

_HIGH = jax.lax.Precision.HIGHEST


def _bdot(a, b):
    return jax.lax.dot_general(
        a.astype(jnp.bfloat16), b.astype(jnp.bfloat16),
        (((1,), (1,)), ((), ())), preferred_element_type=jnp.float32)


def _k1_body(z_ref, xxt_ref, xxc_ref, w1_ref, b1_ref, idx_ref, h1_ref,
             xy_ref):
    i = pl.program_id(0)

    zb = z_ref[pl.ds(i * BR1, BR1), :]
    zball = z_ref[...]
    xy_ref[...] = _bdot(zb, zball)
    h1_ref[...] = jax.nn.relu(
        jax.lax.dot_general(zb.astype(jnp.bfloat16),
                            w1_ref[...].astype(jnp.bfloat16),
                            (((1,), (0,)), ((), ())),
                            preferred_element_type=jnp.float32) + b1_ref[...])
    xxt = xxt_ref[...]
    big = jnp.int32(1 << 30)
    colio = jax.lax.broadcasted_iota(jnp.int32, (SR, N), 1)
    NW = N // 128
    lane_io = jax.lax.broadcasted_iota(jnp.int32, (SR, 128), 1)

    def _batcher16():
        n, pairs, p = 16, [], 1
        while p < n:
            k = p
            while k >= 1:
                for jj in range(k % p, n - k, 2 * k):
                    for ii in range(0, min(k, n - jj - k)):
                        if (ii + jj) // (2 * p) == (ii + jj + k) // (2 * p):
                            pairs.append((ii + jj, ii + jj + k))
                k //= 2
            p *= 2
        return pairs

    _P16 = _batcher16()
    _BIT16 = [(ii, ii + k) for k in (8, 4, 2, 1)
              for ii in range(16) if (ii % (2 * k)) < k]

    def _sub(j, _):
        xy = xy_ref[pl.ds(j * SR, SR), :]
        xxb = xxc_ref[pl.ds(j * SR, SR), :]
        dist2 = (xxb + xxt) - 2.0 * xy
        dist = jnp.sqrt(jnp.maximum(dist2, 0.0) + 1e-8)
        rowio = (jax.lax.broadcasted_iota(jnp.int32, (SR, N), 0)
                 + i * BR1 + j * SR)
        dall = jnp.where(rowio == colio, jnp.inf, dist)
        V = [jax.lax.slice(dall, (0, w * 128), (SR, (w + 1) * 128))
             for w in range(NW)]
        C = [jnp.full((SR, 128), w, jnp.int32) for w in range(NW)]

        def _ce(a, b):
            va, vb, ca, cb = V[a], V[b], C[a], C[b]
            cond = (va < vb) | ((va == vb) & (ca < cb))
            V[a] = jnp.where(cond, va, vb)
            V[b] = jnp.where(cond, vb, va)
            C[a] = jnp.where(cond, ca, cb)
            C[b] = jnp.where(cond, cb, ca)

        bases = [g * 16 for g in range(NW // 16)]
        for base in bases:
            for a, b in _P16:
                _ce(base + a, base + b)
        while len(bases) > 1:
            nxt = []
            for q in range(0, len(bases), 2):
                ba, bb = bases[q], bases[q + 1]
                for t in range(16):
                    _ce(ba + t, bb + 15 - t)
                nxt.append(ba)
            if len(nxt) > 1:
                for base in nxt:
                    for a, b in _BIT16:
                        _ce(base + a, base + b)
            bases = nxt

        vv = jnp.concatenate(V[:16], axis=1)
        gc = jnp.concatenate([C[t] * 128 + lane_io for t in range(16)],
                             axis=1)
        out = []
        for _ in range(KNN):
            m = jnp.min(vv, axis=1, keepdims=True)
            key = jnp.where(vv == m, gc, big)
            c = jnp.min(key, axis=1, keepdims=True)
            out.append(c)
            vv = jnp.where(key == c, jnp.inf, vv)
        idx_ref[pl.ds(j * SR, SR), :] = jnp.concatenate(out, axis=1)
        return 0

    lax.fori_loop(0, BR1 // SR, _sub, 0)


def _run_k1(z, xxt, xxc, W1, b1):
    return pl.pallas_call(
        _k1_body,
        grid=(G1,),
        in_specs=[
            pl.BlockSpec((N, D), lambda i: (0, 0)),
            pl.BlockSpec((1, N), lambda i: (0, 0)),
            pl.BlockSpec((BR1, 1), lambda i: (i, 0)),
            pl.BlockSpec((D, D), lambda i: (0, 0)),
            pl.BlockSpec((1, D), lambda i: (0, 0)),
        ],
        out_specs=[
            pl.BlockSpec((BR1, KNN), lambda i: (i, 0)),
            pl.BlockSpec((BR1, D), lambda i: (i, 0)),
        ],
        out_shape=[
            jax.ShapeDtypeStruct((N, KNN), jnp.int32),
            jax.ShapeDtypeStruct((N, D), jnp.float32),
        ],
        scratch_shapes=[
            pltpu.VMEM((BR1, N), jnp.float32),
        ],
    )(z, xxt, xxc, W1, b1)


NC = 2
NS = 16
HALF = N // NC
RF = HALF // NS
RV = N // NS
ZCH = 32768
ZITER = HALF * N // NS // ZCH


def _k2_body(idx_hbm, a_hbm, idx_f, idx_r, zbuf, ones, sem):
    c = lax.axis_index("c")
    s = lax.axis_index("s")

    def _zinit(t, _):
        zbuf[pl.ds(t * 16, 16)] = jnp.zeros((16,), jnp.float32)
        return 0
    lax.fori_loop(0, ZCH // 16, _zinit, 0)
    ones[...] = jnp.ones((16,), jnp.float32)

    tile_base = (c * HALF + s * RF) * N

    def _zfill(t, _):
        pltpu.sync_copy(zbuf, a_hbm.at[pl.ds(tile_base + t * ZCH, ZCH)])
        return 0
    lax.fori_loop(0, ZITER, _zfill, 0)

    plsc.subcore_barrier()

    rf_base = c * HALF + s * RF
    pltpu.sync_copy(idx_hbm.at[pl.ds(rf_base * KNN, RF * KNN)], idx_f)
    rr_base = s * RV
    pltpu.sync_copy(idx_hbm.at[pl.ds(rr_base * KNN, RV * KNN)], idx_r)

    safe = rf_base * N + idx_f[pl.ds(0, KNN)]

    def _fwd(g, _):
        ds = []
        for u in range(8):
            r = g * 8 + u
            p = (rf_base + r) * N + idx_f[pl.ds(r * KNN, KNN)]
            ds.append(pltpu.async_copy(ones, a_hbm.at[p], sem))
        for d in ds:
            d.wait()
        return 0
    lax.fori_loop(0, RF // 8, _fwd, 0)

    lo = c * HALF
    hi = lo + HALF

    def _rev(g, _):
        ds = []
        for u in range(8):
            r = g * 8 + u
            v = idx_r[pl.ds(r * KNN, KNN)]
            p = v * N + (rr_base + r)
            keep = (v >= lo) & (v < hi)
            p = jnp.where(keep, p, safe)
            ds.append(pltpu.async_copy(ones, a_hbm.at[p], sem))
        for d in ds:
            d.wait()
        return 0
    lax.fori_loop(0, RV // 8, _rev, 0)


def _run_k2(idx):
    mesh = plsc.VectorSubcoreMesh(core_axis_name="c", subcore_axis_name="s")
    f = functools.partial(
        pl.kernel,
        out_type=jax.ShapeDtypeStruct((N * N,), jnp.float32),
        mesh=mesh,
        scratch_types=[
            pltpu.VMEM((RF * KNN,), jnp.int32),
            pltpu.VMEM((RV * KNN,), jnp.int32),
            pltpu.VMEM((ZCH,), jnp.float32),
            pltpu.VMEM((16,), jnp.float32),
            pltpu.SemaphoreType.DMA,
        ],
    )(_k2_body)
    return f(idx.reshape(N * KNN))


BR3 = 256
CK = 1024
KS = N // CK


def _k3_body(a_ref, h1_ref, w2_ref, b2_ref, out_ref, acc_ref, deg_ref):
    k = pl.program_id(1)

    @pl.when(k == 0)
    def _():
        acc_ref[...] = jnp.zeros_like(acc_ref)
        deg_ref[...] = jnp.zeros_like(deg_ref)

    a = a_ref[...]
    hb = h1_ref[pl.ds(k * CK, CK), :]
    acc_ref[...] += jax.lax.dot_general(
        a.astype(jnp.bfloat16), hb.astype(jnp.bfloat16),
        (((1,), (0,)), ((), ())), preferred_element_type=jnp.float32)
    deg_ref[...] += jnp.sum(a, axis=1, keepdims=True)

    @pl.when(k == KS - 1)
    def _():
        agg = acc_ref[...] / (deg_ref[...] + 1e-6)
        out_ref[...] = jax.nn.relu(
            jax.lax.dot_general(
                agg.astype(jnp.bfloat16), w2_ref[...].astype(jnp.bfloat16),
                (((1,), (0,)), ((), ())),
                preferred_element_type=jnp.float32) + b2_ref[...])


def _run_k3(A, h1, W2, b2):
    return pl.pallas_call(
        _k3_body,
        grid=(N // BR3, KS),
        in_specs=[
            pl.BlockSpec((BR3, CK), lambda i, k: (i, k)),
            pl.BlockSpec((N, D), lambda i, k: (0, 0)),
            pl.BlockSpec((D, D), lambda i, k: (0, 0)),
            pl.BlockSpec((1, D), lambda i, k: (0, 0)),
        ],
        out_specs=pl.BlockSpec((BR3, D), lambda i, k: (i, 0)),
        out_shape=jax.ShapeDtypeStruct((N, D), jnp.float32),
        scratch_shapes=[
            pltpu.VMEM((BR3, D), jnp.float32),
            pltpu.VMEM((BR3, 1), jnp.float32),
        ],
        compiler_params=pltpu.CompilerParams(
            dimension_semantics=("parallel", "arbitrary")),
    )(A, h1, W2, b2)


def kernel(z, W1, b1, W2, b2):
    xxc = jnp.sum(z ** 2, axis=1, keepdims=True)
    xxt = xxc.reshape(1, N)
    idx, h1 = _run_k1(z, xxt, xxc, W1, b1.reshape(1, D))
    a_flat = _run_k2(idx)
    A = a_flat.reshape(N, N)
    h = _run_k3(A, h1, W2, b2.reshape(1, D))
    return (h, A)


# speedup vs baseline: 3.9099x; 1.5233x over previous
import functools

import jax
import jax.numpy as jnp
from jax import lax
from jax.experimental import pallas as pl
from jax.experimental.pallas import tpu as pltpu
from jax.experimental.pallas import tpu_sc as plsc

N = 8192
D = 256
KNN = 16


BR1 = 256
SR = 8
G1 = N // BR1

_HIGH = jax.lax.Precision.HIGHEST


def _bdot(a, b):
    return jax.lax.dot_general(
        a.astype(jnp.bfloat16), b.astype(jnp.bfloat16),
        (((1,), (1,)), ((), ())), preferred_element_type=jnp.float32)


def _k1_body(z_ref, xxt_ref, xxc_ref, w1_ref, b1_ref, idx_ref, h1_ref,
             xy_ref):
    i = pl.program_id(0)

    zb = z_ref[pl.ds(i * BR1, BR1), :]
    zball = z_ref[...]
    xy_ref[...] = _bdot(zb, zball)
    h1_ref[...] = jax.nn.relu(
        jax.lax.dot_general(zb.astype(jnp.bfloat16),
                            w1_ref[...].astype(jnp.bfloat16),
                            (((1,), (0,)), ((), ())),
                            preferred_element_type=jnp.float32) + b1_ref[...])
    xxt = xxt_ref[...]
    big = jnp.int32(1 << 30)
    colio = jax.lax.broadcasted_iota(jnp.int32, (SR, N), 1)
    NW = N // 128
    lane_io = jax.lax.broadcasted_iota(jnp.int32, (SR, 128), 1)

    def _batcher16():
        n, pairs, p = 16, [], 1
        while p < n:
            k = p
            while k >= 1:
                for jj in range(k % p, n - k, 2 * k):
                    for ii in range(0, min(k, n - jj - k)):
                        if (ii + jj) // (2 * p) == (ii + jj + k) // (2 * p):
                            pairs.append((ii + jj, ii + jj + k))
                k //= 2
            p *= 2
        return pairs

    _P16 = _batcher16()
    _BIT16 = [(ii, ii + k) for k in (8, 4, 2, 1)
              for ii in range(16) if (ii % (2 * k)) < k]

    def _sub(j, _):
        xy = xy_ref[pl.ds(j * SR, SR), :]
        xxb = xxc_ref[pl.ds(j * SR, SR), :]
        dist2 = (xxb + xxt) - 2.0 * xy
        dist = jnp.sqrt(jnp.maximum(dist2, 0.0) + 1e-8)
        rowio = (jax.lax.broadcasted_iota(jnp.int32, (SR, N), 0)
                 + i * BR1 + j * SR)
        dall = jnp.where(rowio == colio, jnp.inf, dist)
        V = [jax.lax.slice(dall, (0, w * 128), (SR, (w + 1) * 128))
             for w in range(NW)]
        C = [jnp.full((SR, 128), w, jnp.int32) for w in range(NW)]

        def _ce(a, b):
            va, vb, ca, cb = V[a], V[b], C[a], C[b]
            cond = (va < vb) | ((va == vb) & (ca < cb))
            V[a] = jnp.where(cond, va, vb)
            V[b] = jnp.where(cond, vb, va)
            C[a] = jnp.where(cond, ca, cb)
            C[b] = jnp.where(cond, cb, ca)

        bases = [g * 16 for g in range(NW // 16)]
        for base in bases:
            for a, b in _P16:
                _ce(base + a, base + b)
        while len(bases) > 1:
            nxt = []
            for q in range(0, len(bases), 2):
                ba, bb = bases[q], bases[q + 1]
                for t in range(16):
                    _ce(ba + t, bb + 15 - t)
                nxt.append(ba)
            if len(nxt) > 1:
                for base in nxt:
                    for a, b in _BIT16:
                        _ce(base + a, base + b)
            bases = nxt

        vv = jnp.concatenate(V[:16], axis=1)
        gc = jnp.concatenate([C[t] * 128 + lane_io for t in range(16)],
                             axis=1)
        out = []
        for _ in range(KNN):
            m = jnp.min(vv, axis=1, keepdims=True)
            key = jnp.where(vv == m, gc, big)
            c = jnp.min(key, axis=1, keepdims=True)
            out.append(c)
            vv = jnp.where(key == c, jnp.inf, vv)
        idx_ref[pl.ds(j * SR, SR), :] = jnp.concatenate(out, axis=1)
        return 0

    lax.fori_loop(0, BR1 // SR, _sub, 0, unroll=2)


def _run_k1(z, xxt, xxc, W1, b1):
    return pl.pallas_call(
        _k1_body,
        grid=(G1,),
        in_specs=[
            pl.BlockSpec((N, D), lambda i: (0, 0)),
            pl.BlockSpec((1, N), lambda i: (0, 0)),
            pl.BlockSpec((BR1, 1), lambda i: (i, 0)),
            pl.BlockSpec((D, D), lambda i: (0, 0)),
            pl.BlockSpec((1, D), lambda i: (0, 0)),
        ],
        out_specs=[
            pl.BlockSpec((BR1, KNN), lambda i: (i, 0)),
            pl.BlockSpec((BR1, D), lambda i: (i, 0)),
        ],
        out_shape=[
            jax.ShapeDtypeStruct((N, KNN), jnp.int32),
            jax.ShapeDtypeStruct((N, D), jnp.float32),
        ],
        scratch_shapes=[
            pltpu.VMEM((BR1, N), jnp.float32),
        ],
    )(z, xxt, xxc, W1, b1)


NC = 2
NS = 16
HALF = N // NC
RF = HALF // NS
RV = N // NS
ZCH = 32768
ZITER = HALF * N // NS // ZCH


def _k2_body(idx_hbm, a_hbm, idx_f, idx_r, zbuf, ones, sem):
    c = lax.axis_index("c")
    s = lax.axis_index("s")

    def _zinit(t, _):
        zbuf[pl.ds(t * 16, 16)] = jnp.zeros((16,), jnp.float32)
        return 0
    lax.fori_loop(0, ZCH // 16, _zinit, 0)
    ones[...] = jnp.ones((16,), jnp.float32)

    tile_base = (c * HALF + s * RF) * N

    def _zfill(t, _):
        pltpu.sync_copy(zbuf, a_hbm.at[pl.ds(tile_base + t * ZCH, ZCH)])
        return 0
    lax.fori_loop(0, ZITER, _zfill, 0)

    plsc.subcore_barrier()

    rf_base = c * HALF + s * RF
    pltpu.sync_copy(idx_hbm.at[pl.ds(rf_base * KNN, RF * KNN)], idx_f)
    rr_base = s * RV
    pltpu.sync_copy(idx_hbm.at[pl.ds(rr_base * KNN, RV * KNN)], idx_r)

    safe = rf_base * N + idx_f[pl.ds(0, KNN)]

    def _fwd(g, _):
        ds = []
        for u in range(8):
            r = g * 8 + u
            p = (rf_base + r) * N + idx_f[pl.ds(r * KNN, KNN)]
            ds.append(pltpu.async_copy(ones, a_hbm.at[p], sem))
        for d in ds:
            d.wait()
        return 0
    lax.fori_loop(0, RF // 8, _fwd, 0)

    lo = c * HALF
    hi = lo + HALF

    def _rev(g, _):
        ds = []
        for u in range(8):
            r = g * 8 + u
            v = idx_r[pl.ds(r * KNN, KNN)]
            p = v * N + (rr_base + r)
            keep = (v >= lo) & (v < hi)
            p = jnp.where(keep, p, safe)
            ds.append(pltpu.async_copy(ones, a_hbm.at[p], sem))
        for d in ds:
            d.wait()
        return 0
    lax.fori_loop(0, RV // 8, _rev, 0)


def _run_k2(idx):
    mesh = plsc.VectorSubcoreMesh(core_axis_name="c", subcore_axis_name="s")
    f = functools.partial(
        pl.kernel,
        out_type=jax.ShapeDtypeStruct((N * N,), jnp.float32),
        mesh=mesh,
        scratch_types=[
            pltpu.VMEM((RF * KNN,), jnp.int32),
            pltpu.VMEM((RV * KNN,), jnp.int32),
            pltpu.VMEM((ZCH,), jnp.float32),
            pltpu.VMEM((16,), jnp.float32),
            pltpu.SemaphoreType.DMA,
        ],
    )(_k2_body)
    return f(idx.reshape(N * KNN))


BR3 = 256
CK = 1024
KS = N // CK


def _k3_body(a_ref, h1_ref, w2_ref, b2_ref, out_ref, acc_ref, deg_ref):
    k = pl.program_id(1)

    @pl.when(k == 0)
    def _():
        acc_ref[...] = jnp.zeros_like(acc_ref)
        deg_ref[...] = jnp.zeros_like(deg_ref)

    a = a_ref[...]
    hb = h1_ref[pl.ds(k * CK, CK), :]
    acc_ref[...] += jax.lax.dot_general(
        a.astype(jnp.bfloat16), hb.astype(jnp.bfloat16),
        (((1,), (0,)), ((), ())), preferred_element_type=jnp.float32)
    deg_ref[...] += jnp.sum(a, axis=1, keepdims=True)

    @pl.when(k == KS - 1)
    def _():
        agg = acc_ref[...] / (deg_ref[...] + 1e-6)
        out_ref[...] = jax.nn.relu(
            jax.lax.dot_general(
                agg.astype(jnp.bfloat16), w2_ref[...].astype(jnp.bfloat16),
                (((1,), (0,)), ((), ())),
                preferred_element_type=jnp.float32) + b2_ref[...])


def _run_k3(A, h1, W2, b2):
    return pl.pallas_call(
        _k3_body,
        grid=(N // BR3, KS),
        in_specs=[
            pl.BlockSpec((BR3, CK), lambda i, k: (i, k)),
            pl.BlockSpec((N, D), lambda i, k: (0, 0)),
            pl.BlockSpec((D, D), lambda i, k: (0, 0)),
            pl.BlockSpec((1, D), lambda i, k: (0, 0)),
        ],
        out_specs=pl.BlockSpec((BR3, D), lambda i, k: (i, 0)),
        out_shape=jax.ShapeDtypeStruct((N, D), jnp.float32),
        scratch_shapes=[
            pltpu.VMEM((BR3, D), jnp.float32),
            pltpu.VMEM((BR3, 1), jnp.float32),
        ],
        compiler_params=pltpu.CompilerParams(
            dimension_semantics=("parallel", "arbitrary")),
    )(A, h1, W2, b2)


def kernel(z, W1, b1, W2, b2):
    xxc = jnp.sum(z ** 2, axis=1, keepdims=True)
    xxt = xxc.reshape(1, N)
    idx, h1 = _run_k1(z, xxt, xxc, W1, b1.reshape(1, D))
    a_flat = _run_k2(idx)
    A = a_flat.reshape(N, N)
    h = _run_k3(A, h1, W2, b2.reshape(1, D))
    return (h, A)


# speedup vs baseline: 5.3200x; 1.3607x over previous
import functools

import jax
import jax.numpy as jnp
from jax import lax
from jax.experimental import pallas as pl
from jax.experimental.pallas import tpu as pltpu
from jax.experimental.pallas import tpu_sc as plsc

N = 8192
D = 256
KNN = 16


BR1 = 256
SR = 8
G1 = N // BR1

_HIGH = jax.lax.Precision.HIGHEST


def _bdot(a, b):
    return jax.lax.dot_general(
        a.astype(jnp.bfloat16), b.astype(jnp.bfloat16),
        (((1,), (1,)), ((), ())), preferred_element_type=jnp.float32)


def _k1_body(z_ref, xxt_ref, xxc_ref, w1_ref, b1_ref, idx_ref, h1_ref,
             xy_ref):
    i = pl.program_id(0)

    zb = z_ref[pl.ds(i * BR1, BR1), :]
    zball = z_ref[...]
    xy_ref[...] = _bdot(zb, zball)
    h1_ref[...] = jax.nn.relu(
        jax.lax.dot_general(zb.astype(jnp.bfloat16),
                            w1_ref[...].astype(jnp.bfloat16),
                            (((1,), (0,)), ((), ())),
                            preferred_element_type=jnp.float32) + b1_ref[...])
    xxt = xxt_ref[...]
    big = jnp.int32(1 << 30)
    colio = jax.lax.broadcasted_iota(jnp.int32, (SR, N), 1)
    NW = N // 128
    lane_io = jax.lax.broadcasted_iota(jnp.int32, (SR, 128), 1)

    def _batcher16():
        n, pairs, p = 16, [], 1
        while p < n:
            k = p
            while k >= 1:
                for jj in range(k % p, n - k, 2 * k):
                    for ii in range(0, min(k, n - jj - k)):
                        if (ii + jj) // (2 * p) == (ii + jj + k) // (2 * p):
                            pairs.append((ii + jj, ii + jj + k))
                k //= 2
            p *= 2
        return pairs

    _P16 = _batcher16()
    _BIT16 = [(ii, ii + k) for k in (8, 4, 2, 1)
              for ii in range(16) if (ii % (2 * k)) < k]

    def _sub(j, _):
        xy = xy_ref[pl.ds(j * SR, SR), :]
        xxb = xxc_ref[pl.ds(j * SR, SR), :]
        dist2 = (xxb + xxt) - 2.0 * xy
        dist = jnp.sqrt(jnp.maximum(dist2, 0.0) + 1e-8)
        rowio = (jax.lax.broadcasted_iota(jnp.int32, (SR, N), 0)
                 + i * BR1 + j * SR)
        dall = jnp.where(rowio == colio, jnp.inf, dist)
        V = [jax.lax.slice(dall, (0, w * 128), (SR, (w + 1) * 128))
             for w in range(NW)]
        C = [jnp.full((SR, 128), w, jnp.int32) for w in range(NW)]

        def _ce(a, b):
            va, vb, ca, cb = V[a], V[b], C[a], C[b]
            cond = (va < vb) | ((va == vb) & (ca < cb))
            V[a] = jnp.where(cond, va, vb)
            V[b] = jnp.where(cond, vb, va)
            C[a] = jnp.where(cond, ca, cb)
            C[b] = jnp.where(cond, cb, ca)

        bases = [g * 16 for g in range(NW // 16)]
        for base in bases:
            for a, b in _P16:
                _ce(base + a, base + b)
        while len(bases) > 1:
            nxt = []
            for q in range(0, len(bases), 2):
                ba, bb = bases[q], bases[q + 1]
                for t in range(16):
                    _ce(ba + t, bb + 15 - t)
                nxt.append(ba)
            if len(nxt) > 1:
                for base in nxt:
                    for a, b in _BIT16:
                        _ce(base + a, base + b)
            bases = nxt

        vv = jnp.concatenate(V[:16], axis=1)
        gc = jnp.concatenate([C[t] * 128 + lane_io for t in range(16)],
                             axis=1)
        out = []
        for _ in range(KNN):
            m = jnp.min(vv, axis=1, keepdims=True)
            key = jnp.where(vv == m, gc, big)
            c = jnp.min(key, axis=1, keepdims=True)
            out.append(c)
            vv = jnp.where(key == c, jnp.inf, vv)
        idx_ref[pl.ds(j * SR, SR), :] = jnp.concatenate(out, axis=1)
        return 0

    lax.fori_loop(0, BR1 // SR, _sub, 0, unroll=4)


def _run_k1(z, xxt, xxc, W1, b1):
    return pl.pallas_call(
        _k1_body,
        grid=(G1,),
        in_specs=[
            pl.BlockSpec((N, D), lambda i: (0, 0)),
            pl.BlockSpec((1, N), lambda i: (0, 0)),
            pl.BlockSpec((BR1, 1), lambda i: (i, 0)),
            pl.BlockSpec((D, D), lambda i: (0, 0)),
            pl.BlockSpec((1, D), lambda i: (0, 0)),
        ],
        out_specs=[
            pl.BlockSpec((BR1, KNN), lambda i: (i, 0)),
            pl.BlockSpec((BR1, D), lambda i: (i, 0)),
        ],
        out_shape=[
            jax.ShapeDtypeStruct((N, KNN), jnp.int32),
            jax.ShapeDtypeStruct((N, D), jnp.float32),
        ],
        scratch_shapes=[
            pltpu.VMEM((BR1, N), jnp.float32),
        ],
    )(z, xxt, xxc, W1, b1)


NC = 2
NS = 16
HALF = N // NC
RF = HALF // NS
RV = N // NS
ZCH = 32768
ZITER = HALF * N // NS // ZCH


def _k2_body(idx_hbm, a_hbm, idx_f, idx_r, zbuf, ones, sem):
    c = lax.axis_index("c")
    s = lax.axis_index("s")

    def _zinit(t, _):
        zbuf[pl.ds(t * 16, 16)] = jnp.zeros((16,), jnp.float32)
        return 0
    lax.fori_loop(0, ZCH // 16, _zinit, 0)
    ones[...] = jnp.ones((16,), jnp.float32)

    tile_base = (c * HALF + s * RF) * N

    def _zfill(t, _):
        pltpu.sync_copy(zbuf, a_hbm.at[pl.ds(tile_base + t * ZCH, ZCH)])
        return 0
    lax.fori_loop(0, ZITER, _zfill, 0)

    plsc.subcore_barrier()

    rf_base = c * HALF + s * RF
    pltpu.sync_copy(idx_hbm.at[pl.ds(rf_base * KNN, RF * KNN)], idx_f)
    rr_base = s * RV
    pltpu.sync_copy(idx_hbm.at[pl.ds(rr_base * KNN, RV * KNN)], idx_r)

    safe = rf_base * N + idx_f[pl.ds(0, KNN)]

    def _fwd(g, _):
        ds = []
        for u in range(8):
            r = g * 8 + u
            p = (rf_base + r) * N + idx_f[pl.ds(r * KNN, KNN)]
            ds.append(pltpu.async_copy(ones, a_hbm.at[p], sem))
        for d in ds:
            d.wait()
        return 0
    lax.fori_loop(0, RF // 8, _fwd, 0)

    lo = c * HALF
    hi = lo + HALF

    def _rev(g, _):
        ds = []
        for u in range(8):
            r = g * 8 + u
            v = idx_r[pl.ds(r * KNN, KNN)]
            p = v * N + (rr_base + r)
            keep = (v >= lo) & (v < hi)
            p = jnp.where(keep, p, safe)
            ds.append(pltpu.async_copy(ones, a_hbm.at[p], sem))
        for d in ds:
            d.wait()
        return 0
    lax.fori_loop(0, RV // 8, _rev, 0)


def _run_k2(idx):
    mesh = plsc.VectorSubcoreMesh(core_axis_name="c", subcore_axis_name="s")
    f = functools.partial(
        pl.kernel,
        out_type=jax.ShapeDtypeStruct((N * N,), jnp.float32),
        mesh=mesh,
        scratch_types=[
            pltpu.VMEM((RF * KNN,), jnp.int32),
            pltpu.VMEM((RV * KNN,), jnp.int32),
            pltpu.VMEM((ZCH,), jnp.float32),
            pltpu.VMEM((16,), jnp.float32),
            pltpu.SemaphoreType.DMA,
        ],
    )(_k2_body)
    return f(idx.reshape(N * KNN))


BR3 = 256
CK = 1024
KS = N // CK


def _k3_body(a_ref, h1_ref, w2_ref, b2_ref, out_ref, acc_ref, deg_ref):
    k = pl.program_id(1)

    @pl.when(k == 0)
    def _():
        acc_ref[...] = jnp.zeros_like(acc_ref)
        deg_ref[...] = jnp.zeros_like(deg_ref)

    a = a_ref[...]
    hb = h1_ref[pl.ds(k * CK, CK), :]
    acc_ref[...] += jax.lax.dot_general(
        a.astype(jnp.bfloat16), hb.astype(jnp.bfloat16),
        (((1,), (0,)), ((), ())), preferred_element_type=jnp.float32)
    deg_ref[...] += jnp.sum(a, axis=1, keepdims=True)

    @pl.when(k == KS - 1)
    def _():
        agg = acc_ref[...] / (deg_ref[...] + 1e-6)
        out_ref[...] = jax.nn.relu(
            jax.lax.dot_general(
                agg.astype(jnp.bfloat16), w2_ref[...].astype(jnp.bfloat16),
                (((1,), (0,)), ((), ())),
                preferred_element_type=jnp.float32) + b2_ref[...])


def _run_k3(A, h1, W2, b2):
    return pl.pallas_call(
        _k3_body,
        grid=(N // BR3, KS),
        in_specs=[
            pl.BlockSpec((BR3, CK), lambda i, k: (i, k)),
            pl.BlockSpec((N, D), lambda i, k: (0, 0)),
            pl.BlockSpec((D, D), lambda i, k: (0, 0)),
            pl.BlockSpec((1, D), lambda i, k: (0, 0)),
        ],
        out_specs=pl.BlockSpec((BR3, D), lambda i, k: (i, 0)),
        out_shape=jax.ShapeDtypeStruct((N, D), jnp.float32),
        scratch_shapes=[
            pltpu.VMEM((BR3, D), jnp.float32),
            pltpu.VMEM((BR3, 1), jnp.float32),
        ],
        compiler_params=pltpu.CompilerParams(
            dimension_semantics=("parallel", "arbitrary")),
    )(A, h1, W2, b2)


def kernel(z, W1, b1, W2, b2):
    xxc = jnp.sum(z ** 2, axis=1, keepdims=True)
    xxt = xxc.reshape(1, N)
    idx, h1 = _run_k1(z, xxt, xxc, W1, b1.reshape(1, D))
    a_flat = _run_k2(idx)
    A = a_flat.reshape(N, N)
    h = _run_k3(A, h1, W2, b2.reshape(1, D))
    return (h, A)


# speedup vs baseline: 6.4449x; 1.2114x over previous
import functools

import jax
import jax.numpy as jnp
from jax import lax
from jax.experimental import pallas as pl
from jax.experimental.pallas import tpu as pltpu
from jax.experimental.pallas import tpu_sc as plsc

N = 8192
D = 256
KNN = 16


BR1 = 256
SR = 8
G1 = N // BR1

_HIGH = jax.lax.Precision.HIGHEST


def _bdot(a, b):
    return jax.lax.dot_general(
        a.astype(jnp.bfloat16), b.astype(jnp.bfloat16),
        (((1,), (1,)), ((), ())), preferred_element_type=jnp.float32)


def _k1_body(z_ref, xxt_ref, xxc_ref, w1_ref, b1_ref, idx_ref, h1_ref,
             xy_ref):
    i = pl.program_id(0)

    zb = z_ref[pl.ds(i * BR1, BR1), :]
    zball = z_ref[...]
    xy_ref[...] = _bdot(zb, zball)
    h1_ref[...] = jax.nn.relu(
        jax.lax.dot_general(zb.astype(jnp.bfloat16),
                            w1_ref[...].astype(jnp.bfloat16),
                            (((1,), (0,)), ((), ())),
                            preferred_element_type=jnp.float32) + b1_ref[...])
    xxt = xxt_ref[...]
    big = jnp.int32(1 << 30)
    colio = jax.lax.broadcasted_iota(jnp.int32, (SR, N), 1)
    NW = N // 128
    lane_io = jax.lax.broadcasted_iota(jnp.int32, (SR, 128), 1)

    def _batcher16():
        n, pairs, p = 16, [], 1
        while p < n:
            k = p
            while k >= 1:
                for jj in range(k % p, n - k, 2 * k):
                    for ii in range(0, min(k, n - jj - k)):
                        if (ii + jj) // (2 * p) == (ii + jj + k) // (2 * p):
                            pairs.append((ii + jj, ii + jj + k))
                k //= 2
            p *= 2
        return pairs

    _P16 = _batcher16()
    _BIT16 = [(ii, ii + k) for k in (8, 4, 2, 1)
              for ii in range(16) if (ii % (2 * k)) < k]

    def _sub(j, _):
        xy = xy_ref[pl.ds(j * SR, SR), :]
        xxb = xxc_ref[pl.ds(j * SR, SR), :]
        dist2 = (xxb + xxt) - 2.0 * xy
        dist = jnp.sqrt(jnp.maximum(dist2, 0.0) + 1e-8)
        rowio = (jax.lax.broadcasted_iota(jnp.int32, (SR, N), 0)
                 + i * BR1 + j * SR)
        dall = jnp.where(rowio == colio, jnp.inf, dist)
        V = [jax.lax.slice(dall, (0, w * 128), (SR, (w + 1) * 128))
             for w in range(NW)]
        C = [jnp.full((SR, 128), w, jnp.int32) for w in range(NW)]

        def _ce(a, b):
            va, vb, ca, cb = V[a], V[b], C[a], C[b]
            cond = (va < vb) | ((va == vb) & (ca < cb))
            V[a] = jnp.where(cond, va, vb)
            V[b] = jnp.where(cond, vb, va)
            C[a] = jnp.where(cond, ca, cb)
            C[b] = jnp.where(cond, cb, ca)

        bases = [g * 16 for g in range(NW // 16)]
        for base in bases:
            for a, b in _P16:
                _ce(base + a, base + b)
        while len(bases) > 1:
            nxt = []
            for q in range(0, len(bases), 2):
                ba, bb = bases[q], bases[q + 1]
                for t in range(16):
                    _ce(ba + t, bb + 15 - t)
                nxt.append(ba)
            if len(nxt) > 1:
                for base in nxt:
                    for a, b in _BIT16:
                        _ce(base + a, base + b)
            bases = nxt

        vv = jnp.concatenate(V[:16], axis=1)
        gc = jnp.concatenate([C[t] * 128 + lane_io for t in range(16)],
                             axis=1)
        out = []
        for _ in range(KNN):
            m = jnp.min(vv, axis=1, keepdims=True)
            key = jnp.where(vv == m, gc, big)
            c = jnp.min(key, axis=1, keepdims=True)
            out.append(c)
            vv = jnp.where(key == c, jnp.inf, vv)
        idx_ref[pl.ds(j * SR, SR), :] = jnp.concatenate(out, axis=1)
        return 0

    lax.fori_loop(0, BR1 // SR, _sub, 0, unroll=8)


def _run_k1(z, xxt, xxc, W1, b1):
    return pl.pallas_call(
        _k1_body,
        grid=(G1,),
        in_specs=[
            pl.BlockSpec((N, D), lambda i: (0, 0)),
            pl.BlockSpec((1, N), lambda i: (0, 0)),
            pl.BlockSpec((BR1, 1), lambda i: (i, 0)),
            pl.BlockSpec((D, D), lambda i: (0, 0)),
            pl.BlockSpec((1, D), lambda i: (0, 0)),
        ],
        out_specs=[
            pl.BlockSpec((BR1, KNN), lambda i: (i, 0)),
            pl.BlockSpec((BR1, D), lambda i: (i, 0)),
        ],
        out_shape=[
            jax.ShapeDtypeStruct((N, KNN), jnp.int32),
            jax.ShapeDtypeStruct((N, D), jnp.float32),
        ],
        scratch_shapes=[
            pltpu.VMEM((BR1, N), jnp.float32),
        ],
    )(z, xxt, xxc, W1, b1)


NC = 2
NS = 16
HALF = N // NC
RF = HALF // NS
RV = N // NS
ZCH = 32768
ZITER = HALF * N // NS // ZCH


def _k2_body(idx_hbm, a_hbm, idx_f, idx_r, zbuf, ones, sem):
    c = lax.axis_index("c")
    s = lax.axis_index("s")

    def _zinit(t, _):
        zbuf[pl.ds(t * 16, 16)] = jnp.zeros((16,), jnp.float32)
        return 0
    lax.fori_loop(0, ZCH // 16, _zinit, 0)
    ones[...] = jnp.ones((16,), jnp.float32)

    tile_base = (c * HALF + s * RF) * N

    def _zfill(t, _):
        pltpu.sync_copy(zbuf, a_hbm.at[pl.ds(tile_base + t * ZCH, ZCH)])
        return 0
    lax.fori_loop(0, ZITER, _zfill, 0)

    plsc.subcore_barrier()

    rf_base = c * HALF + s * RF
    pltpu.sync_copy(idx_hbm.at[pl.ds(rf_base * KNN, RF * KNN)], idx_f)
    rr_base = s * RV
    pltpu.sync_copy(idx_hbm.at[pl.ds(rr_base * KNN, RV * KNN)], idx_r)

    safe = rf_base * N + idx_f[pl.ds(0, KNN)]

    def _fwd(g, _):
        ds = []
        for u in range(8):
            r = g * 8 + u
            p = (rf_base + r) * N + idx_f[pl.ds(r * KNN, KNN)]
            ds.append(pltpu.async_copy(ones, a_hbm.at[p], sem))
        for d in ds:
            d.wait()
        return 0
    lax.fori_loop(0, RF // 8, _fwd, 0)

    lo = c * HALF
    hi = lo + HALF

    def _rev(g, _):
        ds = []
        for u in range(8):
            r = g * 8 + u
            v = idx_r[pl.ds(r * KNN, KNN)]
            p = v * N + (rr_base + r)
            keep = (v >= lo) & (v < hi)
            p = jnp.where(keep, p, safe)
            ds.append(pltpu.async_copy(ones, a_hbm.at[p], sem))
        for d in ds:
            d.wait()
        return 0
    lax.fori_loop(0, RV // 8, _rev, 0)


def _run_k2(idx):
    mesh = plsc.VectorSubcoreMesh(core_axis_name="c", subcore_axis_name="s")
    f = functools.partial(
        pl.kernel,
        out_type=jax.ShapeDtypeStruct((N * N,), jnp.float32),
        mesh=mesh,
        scratch_types=[
            pltpu.VMEM((RF * KNN,), jnp.int32),
            pltpu.VMEM((RV * KNN,), jnp.int32),
            pltpu.VMEM((ZCH,), jnp.float32),
            pltpu.VMEM((16,), jnp.float32),
            pltpu.SemaphoreType.DMA,
        ],
    )(_k2_body)
    return f(idx.reshape(N * KNN))


BR3 = 256
CK = 1024
KS = N // CK


def _k3_body(a_ref, h1_ref, w2_ref, b2_ref, out_ref, acc_ref, deg_ref):
    k = pl.program_id(1)

    @pl.when(k == 0)
    def _():
        acc_ref[...] = jnp.zeros_like(acc_ref)
        deg_ref[...] = jnp.zeros_like(deg_ref)

    a = a_ref[...]
    hb = h1_ref[pl.ds(k * CK, CK), :]
    acc_ref[...] += jax.lax.dot_general(
        a.astype(jnp.bfloat16), hb.astype(jnp.bfloat16),
        (((1,), (0,)), ((), ())), preferred_element_type=jnp.float32)
    deg_ref[...] += jnp.sum(a, axis=1, keepdims=True)

    @pl.when(k == KS - 1)
    def _():
        agg = acc_ref[...] / (deg_ref[...] + 1e-6)
        out_ref[...] = jax.nn.relu(
            jax.lax.dot_general(
                agg.astype(jnp.bfloat16), w2_ref[...].astype(jnp.bfloat16),
                (((1,), (0,)), ((), ())),
                preferred_element_type=jnp.float32) + b2_ref[...])


def _run_k3(A, h1, W2, b2):
    return pl.pallas_call(
        _k3_body,
        grid=(N // BR3, KS),
        in_specs=[
            pl.BlockSpec((BR3, CK), lambda i, k: (i, k)),
            pl.BlockSpec((N, D), lambda i, k: (0, 0)),
            pl.BlockSpec((D, D), lambda i, k: (0, 0)),
            pl.BlockSpec((1, D), lambda i, k: (0, 0)),
        ],
        out_specs=pl.BlockSpec((BR3, D), lambda i, k: (i, 0)),
        out_shape=jax.ShapeDtypeStruct((N, D), jnp.float32),
        scratch_shapes=[
            pltpu.VMEM((BR3, D), jnp.float32),
            pltpu.VMEM((BR3, 1), jnp.float32),
        ],
        compiler_params=pltpu.CompilerParams(
            dimension_semantics=("parallel", "arbitrary")),
    )(A, h1, W2, b2)


def kernel(z, W1, b1, W2, b2):
    xxc = jnp.sum(z ** 2, axis=1, keepdims=True)
    xxt = xxc.reshape(1, N)
    idx, h1 = _run_k1(z, xxt, xxc, W1, b1.reshape(1, D))
    a_flat = _run_k2(idx)
    A = a_flat.reshape(N, N)
    h = _run_k3(A, h1, W2, b2.reshape(1, D))
    return (h, A)


# speedup vs baseline: 6.9746x; 1.0822x over previous
import functools

import jax
import jax.numpy as jnp
from jax import lax
from jax.experimental import pallas as pl
from jax.experimental.pallas import tpu as pltpu
from jax.experimental.pallas import tpu_sc as plsc

N = 8192
D = 256
KNN = 16


BR1 = 256
SR = 8
G1 = N // BR1

_HIGH = jax.lax.Precision.HIGHEST


def _bdot(a, b):
    return jax.lax.dot_general(
        a.astype(jnp.bfloat16), b.astype(jnp.bfloat16),
        (((1,), (1,)), ((), ())), preferred_element_type=jnp.float32)


def _k1_body(z_ref, xxt_ref, xxc_ref, w1_ref, b1_ref, idx_ref, h1_ref,
             xy_ref):
    i = pl.program_id(0)

    zb = z_ref[pl.ds(i * BR1, BR1), :]
    zball = z_ref[...]
    xy_ref[...] = _bdot(zb, zball)
    h1_ref[...] = jax.nn.relu(
        jax.lax.dot_general(zb.astype(jnp.bfloat16),
                            w1_ref[...].astype(jnp.bfloat16),
                            (((1,), (0,)), ((), ())),
                            preferred_element_type=jnp.float32) + b1_ref[...])
    xxt = xxt_ref[...]
    big = jnp.int32(1 << 30)
    colio = jax.lax.broadcasted_iota(jnp.int32, (SR, N), 1)
    NW = N // 128
    lane_io = jax.lax.broadcasted_iota(jnp.int32, (SR, 128), 1)

    def _batcher16():
        n, pairs, p = 16, [], 1
        while p < n:
            k = p
            while k >= 1:
                for jj in range(k % p, n - k, 2 * k):
                    for ii in range(0, min(k, n - jj - k)):
                        if (ii + jj) // (2 * p) == (ii + jj + k) // (2 * p):
                            pairs.append((ii + jj, ii + jj + k))
                k //= 2
            p *= 2
        return pairs

    _P16 = _batcher16()
    _BIT16 = [(ii, ii + k) for k in (8, 4, 2, 1)
              for ii in range(16) if (ii % (2 * k)) < k]

    def _sub(j, _):
        xy = xy_ref[pl.ds(j * SR, SR), :]
        xxb = xxc_ref[pl.ds(j * SR, SR), :]
        dist2 = (xxb + xxt) - 2.0 * xy
        dist = jnp.sqrt(jnp.maximum(dist2, 0.0) + 1e-8)
        rowio = (jax.lax.broadcasted_iota(jnp.int32, (SR, N), 0)
                 + i * BR1 + j * SR)
        dall = jnp.where(rowio == colio, jnp.inf, dist)
        V = [jax.lax.slice(dall, (0, w * 128), (SR, (w + 1) * 128))
             for w in range(NW)]
        C = [jnp.full((SR, 128), w, jnp.int32) for w in range(NW)]

        def _ce(a, b):
            va, vb, ca, cb = V[a], V[b], C[a], C[b]
            cond = (va < vb) | ((va == vb) & (ca < cb))
            V[a] = jnp.where(cond, va, vb)
            V[b] = jnp.where(cond, vb, va)
            C[a] = jnp.where(cond, ca, cb)
            C[b] = jnp.where(cond, cb, ca)

        bases = [g * 16 for g in range(NW // 16)]
        for base in bases:
            for a, b in _P16:
                _ce(base + a, base + b)
        while len(bases) > 1:
            nxt = []
            for q in range(0, len(bases), 2):
                ba, bb = bases[q], bases[q + 1]
                for t in range(16):
                    _ce(ba + t, bb + 15 - t)
                nxt.append(ba)
            if len(nxt) > 1:
                for base in nxt:
                    for a, b in _BIT16:
                        _ce(base + a, base + b)
            bases = nxt

        vv = jnp.concatenate(V[:16], axis=1)
        gc = jnp.concatenate([C[t] * 128 + lane_io for t in range(16)],
                             axis=1)
        out = []
        for _ in range(KNN):
            m = jnp.min(vv, axis=1, keepdims=True)
            key = jnp.where(vv == m, gc, big)
            c = jnp.min(key, axis=1, keepdims=True)
            out.append(c)
            vv = jnp.where(key == c, jnp.inf, vv)
        idx_ref[pl.ds(j * SR, SR), :] = jnp.concatenate(out, axis=1)
        return 0

    lax.fori_loop(0, BR1 // SR, _sub, 0, unroll=16)


def _run_k1(z, xxt, xxc, W1, b1):
    return pl.pallas_call(
        _k1_body,
        grid=(G1,),
        in_specs=[
            pl.BlockSpec((N, D), lambda i: (0, 0)),
            pl.BlockSpec((1, N), lambda i: (0, 0)),
            pl.BlockSpec((BR1, 1), lambda i: (i, 0)),
            pl.BlockSpec((D, D), lambda i: (0, 0)),
            pl.BlockSpec((1, D), lambda i: (0, 0)),
        ],
        out_specs=[
            pl.BlockSpec((BR1, KNN), lambda i: (i, 0)),
            pl.BlockSpec((BR1, D), lambda i: (i, 0)),
        ],
        out_shape=[
            jax.ShapeDtypeStruct((N, KNN), jnp.int32),
            jax.ShapeDtypeStruct((N, D), jnp.float32),
        ],
        scratch_shapes=[
            pltpu.VMEM((BR1, N), jnp.float32),
        ],
    )(z, xxt, xxc, W1, b1)


NC = 2
NS = 16
HALF = N // NC
RF = HALF // NS
RV = N // NS
ZCH = 32768
ZITER = HALF * N // NS // ZCH


def _k2_body(idx_hbm, a_hbm, idx_f, idx_r, zbuf, ones, sem):
    c = lax.axis_index("c")
    s = lax.axis_index("s")

    def _zinit(t, _):
        zbuf[pl.ds(t * 16, 16)] = jnp.zeros((16,), jnp.float32)
        return 0
    lax.fori_loop(0, ZCH // 16, _zinit, 0)
    ones[...] = jnp.ones((16,), jnp.float32)

    tile_base = (c * HALF + s * RF) * N

    def _zfill(t, _):
        pltpu.sync_copy(zbuf, a_hbm.at[pl.ds(tile_base + t * ZCH, ZCH)])
        return 0
    lax.fori_loop(0, ZITER, _zfill, 0)

    plsc.subcore_barrier()

    rf_base = c * HALF + s * RF
    pltpu.sync_copy(idx_hbm.at[pl.ds(rf_base * KNN, RF * KNN)], idx_f)
    rr_base = s * RV
    pltpu.sync_copy(idx_hbm.at[pl.ds(rr_base * KNN, RV * KNN)], idx_r)

    safe = rf_base * N + idx_f[pl.ds(0, KNN)]

    def _fwd(g, _):
        ds = []
        for u in range(8):
            r = g * 8 + u
            p = (rf_base + r) * N + idx_f[pl.ds(r * KNN, KNN)]
            ds.append(pltpu.async_copy(ones, a_hbm.at[p], sem))
        for d in ds:
            d.wait()
        return 0
    lax.fori_loop(0, RF // 8, _fwd, 0)

    lo = c * HALF
    hi = lo + HALF

    def _rev(g, _):
        ds = []
        for u in range(8):
            r = g * 8 + u
            v = idx_r[pl.ds(r * KNN, KNN)]
            p = v * N + (rr_base + r)
            keep = (v >= lo) & (v < hi)
            p = jnp.where(keep, p, safe)
            ds.append(pltpu.async_copy(ones, a_hbm.at[p], sem))
        for d in ds:
            d.wait()
        return 0
    lax.fori_loop(0, RV // 8, _rev, 0)


def _run_k2(idx):
    mesh = plsc.VectorSubcoreMesh(core_axis_name="c", subcore_axis_name="s")
    f = functools.partial(
        pl.kernel,
        out_type=jax.ShapeDtypeStruct((N * N,), jnp.float32),
        mesh=mesh,
        scratch_types=[
            pltpu.VMEM((RF * KNN,), jnp.int32),
            pltpu.VMEM((RV * KNN,), jnp.int32),
            pltpu.VMEM((ZCH,), jnp.float32),
            pltpu.VMEM((16,), jnp.float32),
            pltpu.SemaphoreType.DMA,
        ],
    )(_k2_body)
    return f(idx.reshape(N * KNN))


BR3 = 256
CK = 1024
KS = N // CK


def _k3_body(a_ref, h1_ref, w2_ref, b2_ref, out_ref, acc_ref, deg_ref):
    k = pl.program_id(1)

    @pl.when(k == 0)
    def _():
        acc_ref[...] = jnp.zeros_like(acc_ref)
        deg_ref[...] = jnp.zeros_like(deg_ref)

    a = a_ref[...]
    hb = h1_ref[pl.ds(k * CK, CK), :]
    acc_ref[...] += jax.lax.dot_general(
        a.astype(jnp.bfloat16), hb.astype(jnp.bfloat16),
        (((1,), (0,)), ((), ())), preferred_element_type=jnp.float32)
    deg_ref[...] += jnp.sum(a, axis=1, keepdims=True)

    @pl.when(k == KS - 1)
    def _():
        agg = acc_ref[...] / (deg_ref[...] + 1e-6)
        out_ref[...] = jax.nn.relu(
            jax.lax.dot_general(
                agg.astype(jnp.bfloat16), w2_ref[...].astype(jnp.bfloat16),
                (((1,), (0,)), ((), ())),
                preferred_element_type=jnp.float32) + b2_ref[...])


def _run_k3(A, h1, W2, b2):
    return pl.pallas_call(
        _k3_body,
        grid=(N // BR3, KS),
        in_specs=[
            pl.BlockSpec((BR3, CK), lambda i, k: (i, k)),
            pl.BlockSpec((N, D), lambda i, k: (0, 0)),
            pl.BlockSpec((D, D), lambda i, k: (0, 0)),
            pl.BlockSpec((1, D), lambda i, k: (0, 0)),
        ],
        out_specs=pl.BlockSpec((BR3, D), lambda i, k: (i, 0)),
        out_shape=jax.ShapeDtypeStruct((N, D), jnp.float32),
        scratch_shapes=[
            pltpu.VMEM((BR3, D), jnp.float32),
            pltpu.VMEM((BR3, 1), jnp.float32),
        ],
        compiler_params=pltpu.CompilerParams(
            dimension_semantics=("parallel", "arbitrary")),
    )(A, h1, W2, b2)


def kernel(z, W1, b1, W2, b2):
    xxc = jnp.sum(z ** 2, axis=1, keepdims=True)
    xxt = xxc.reshape(1, N)
    idx, h1 = _run_k1(z, xxt, xxc, W1, b1.reshape(1, D))
    a_flat = _run_k2(idx)
    A = a_flat.reshape(N, N)
    h = _run_k3(A, h1, W2, b2.reshape(1, D))
    return (h, A)


# speedup vs baseline: 7.1222x; 1.0212x over previous
import functools

import jax
import jax.numpy as jnp
from jax import lax
from jax.experimental import pallas as pl
from jax.experimental.pallas import tpu as pltpu
from jax.experimental.pallas import tpu_sc as plsc

N = 8192
D = 256
KNN = 16


BR1 = 256
SR = 8
G1 = N // BR1

_HIGH = jax.lax.Precision.HIGHEST


def _bdot(a, b):
    return jax.lax.dot_general(
        a.astype(jnp.bfloat16), b.astype(jnp.bfloat16),
        (((1,), (1,)), ((), ())), preferred_element_type=jnp.float32)


def _k1_body(z_ref, xxt_ref, xxc_ref, w1_ref, b1_ref, idx_ref, h1_ref,
             xy_ref):
    i = pl.program_id(0)

    zb = z_ref[pl.ds(i * BR1, BR1), :]
    zball = z_ref[...]
    xy_ref[...] = _bdot(zb, zball)
    h1_ref[...] = jax.nn.relu(
        jax.lax.dot_general(zb.astype(jnp.bfloat16),
                            w1_ref[...].astype(jnp.bfloat16),
                            (((1,), (0,)), ((), ())),
                            preferred_element_type=jnp.float32) + b1_ref[...])
    xxt = xxt_ref[...]
    big = jnp.int32(1 << 30)
    colio = jax.lax.broadcasted_iota(jnp.int32, (SR, N), 1)
    NW = N // 128
    lane_io = jax.lax.broadcasted_iota(jnp.int32, (SR, 128), 1)

    def _batcher16():
        n, pairs, p = 16, [], 1
        while p < n:
            k = p
            while k >= 1:
                for jj in range(k % p, n - k, 2 * k):
                    for ii in range(0, min(k, n - jj - k)):
                        if (ii + jj) // (2 * p) == (ii + jj + k) // (2 * p):
                            pairs.append((ii + jj, ii + jj + k))
                k //= 2
            p *= 2
        return pairs

    _P16 = _batcher16()
    _BIT16 = [(ii, ii + k) for k in (8, 4, 2, 1)
              for ii in range(16) if (ii % (2 * k)) < k]

    def _sub(j, _):
        xy = xy_ref[pl.ds(j * SR, SR), :]
        xxb = xxc_ref[pl.ds(j * SR, SR), :]
        dist2 = (xxb + xxt) - 2.0 * xy
        dist = jnp.sqrt(jnp.maximum(dist2, 0.0) + 1e-8)
        rowio = (jax.lax.broadcasted_iota(jnp.int32, (SR, N), 0)
                 + i * BR1 + j * SR)
        dall = jnp.where(rowio == colio, jnp.inf, dist)
        V = [jax.lax.slice(dall, (0, w * 128), (SR, (w + 1) * 128))
             for w in range(NW)]
        C = [jnp.full((SR, 128), w, jnp.int32) for w in range(NW)]

        def _ce(a, b):
            va, vb, ca, cb = V[a], V[b], C[a], C[b]
            cond = (va < vb) | ((va == vb) & (ca < cb))
            V[a] = jnp.where(cond, va, vb)
            V[b] = jnp.where(cond, vb, va)
            C[a] = jnp.where(cond, ca, cb)
            C[b] = jnp.where(cond, cb, ca)

        bases = [g * 16 for g in range(NW // 16)]
        for base in bases:
            for a, b in _P16:
                _ce(base + a, base + b)
        while len(bases) > 1:
            nxt = []
            for q in range(0, len(bases), 2):
                ba, bb = bases[q], bases[q + 1]
                for t in range(16):
                    _ce(ba + t, bb + 15 - t)
                nxt.append(ba)
            if len(nxt) > 1:
                for base in nxt:
                    for a, b in _BIT16:
                        _ce(base + a, base + b)
            bases = nxt

        vv = jnp.concatenate(V[:16], axis=1)
        gc = jnp.concatenate([C[t] * 128 + lane_io for t in range(16)],
                             axis=1)
        out = []
        for _ in range(KNN):
            m = jnp.min(vv, axis=1, keepdims=True)
            key = jnp.where(vv == m, gc, big)
            c = jnp.min(key, axis=1, keepdims=True)
            out.append(c)
            vv = jnp.where(key == c, jnp.inf, vv)
        idx_ref[pl.ds(j * SR, SR), :] = jnp.concatenate(out, axis=1)
        return 0

    lax.fori_loop(0, BR1 // SR, _sub, 0, unroll=32)


def _run_k1(z, xxt, xxc, W1, b1):
    return pl.pallas_call(
        _k1_body,
        grid=(G1,),
        in_specs=[
            pl.BlockSpec((N, D), lambda i: (0, 0)),
            pl.BlockSpec((1, N), lambda i: (0, 0)),
            pl.BlockSpec((BR1, 1), lambda i: (i, 0)),
            pl.BlockSpec((D, D), lambda i: (0, 0)),
            pl.BlockSpec((1, D), lambda i: (0, 0)),
        ],
        out_specs=[
            pl.BlockSpec((BR1, KNN), lambda i: (i, 0)),
            pl.BlockSpec((BR1, D), lambda i: (i, 0)),
        ],
        out_shape=[
            jax.ShapeDtypeStruct((N, KNN), jnp.int32),
            jax.ShapeDtypeStruct((N, D), jnp.float32),
        ],
        scratch_shapes=[
            pltpu.VMEM((BR1, N), jnp.float32),
        ],
    )(z, xxt, xxc, W1, b1)


NC = 2
NS = 16
HALF = N // NC
RF = HALF // NS
RV = N // NS
ZCH = 32768
ZITER = HALF * N // NS // ZCH


def _k2_body(idx_hbm, a_hbm, idx_f, idx_r, zbuf, ones, sem):
    c = lax.axis_index("c")
    s = lax.axis_index("s")

    def _zinit(t, _):
        zbuf[pl.ds(t * 16, 16)] = jnp.zeros((16,), jnp.float32)
        return 0
    lax.fori_loop(0, ZCH // 16, _zinit, 0)
    ones[...] = jnp.ones((16,), jnp.float32)

    tile_base = (c * HALF + s * RF) * N

    def _zfill(t, _):
        pltpu.sync_copy(zbuf, a_hbm.at[pl.ds(tile_base + t * ZCH, ZCH)])
        return 0
    lax.fori_loop(0, ZITER, _zfill, 0)

    plsc.subcore_barrier()

    rf_base = c * HALF + s * RF
    pltpu.sync_copy(idx_hbm.at[pl.ds(rf_base * KNN, RF * KNN)], idx_f)
    rr_base = s * RV
    pltpu.sync_copy(idx_hbm.at[pl.ds(rr_base * KNN, RV * KNN)], idx_r)

    safe = rf_base * N + idx_f[pl.ds(0, KNN)]

    def _fwd(g, _):
        ds = []
        for u in range(8):
            r = g * 8 + u
            p = (rf_base + r) * N + idx_f[pl.ds(r * KNN, KNN)]
            ds.append(pltpu.async_copy(ones, a_hbm.at[p], sem))
        for d in ds:
            d.wait()
        return 0
    lax.fori_loop(0, RF // 8, _fwd, 0)

    lo = c * HALF
    hi = lo + HALF

    def _rev(g, _):
        ds = []
        for u in range(8):
            r = g * 8 + u
            v = idx_r[pl.ds(r * KNN, KNN)]
            p = v * N + (rr_base + r)
            keep = (v >= lo) & (v < hi)
            p = jnp.where(keep, p, safe)
            ds.append(pltpu.async_copy(ones, a_hbm.at[p], sem))
        for d in ds:
            d.wait()
        return 0
    lax.fori_loop(0, RV // 8, _rev, 0)


def _run_k2(idx):
    mesh = plsc.VectorSubcoreMesh(core_axis_name="c", subcore_axis_name="s")
    f = functools.partial(
        pl.kernel,
        out_type=jax.ShapeDtypeStruct((N * N,), jnp.float32),
        mesh=mesh,
        scratch_types=[
            pltpu.VMEM((RF * KNN,), jnp.int32),
            pltpu.VMEM((RV * KNN,), jnp.int32),
            pltpu.VMEM((ZCH,), jnp.float32),
            pltpu.VMEM((16,), jnp.float32),
            pltpu.SemaphoreType.DMA,
        ],
    )(_k2_body)
    return f(idx.reshape(N * KNN))


BR3 = 256
CK = 1024
KS = N // CK


def _k3_body(a_ref, h1_ref, w2_ref, b2_ref, out_ref, acc_ref, deg_ref):
    k = pl.program_id(1)

    @pl.when(k == 0)
    def _():
        acc_ref[...] = jnp.zeros_like(acc_ref)
        deg_ref[...] = jnp.zeros_like(deg_ref)

    a = a_ref[...]
    hb = h1_ref[pl.ds(k * CK, CK), :]
    acc_ref[...] += jax.lax.dot_general(
        a.astype(jnp.bfloat16), hb.astype(jnp.bfloat16),
        (((1,), (0,)), ((), ())), preferred_element_type=jnp.float32)
    deg_ref[...] += jnp.sum(a, axis=1, keepdims=True)

    @pl.when(k == KS - 1)
    def _():
        agg = acc_ref[...] / (deg_ref[...] + 1e-6)
        out_ref[...] = jax.nn.relu(
            jax.lax.dot_general(
                agg.astype(jnp.bfloat16), w2_ref[...].astype(jnp.bfloat16),
                (((1,), (0,)), ((), ())),
                preferred_element_type=jnp.float32) + b2_ref[...])


def _run_k3(A, h1, W2, b2):
    return pl.pallas_call(
        _k3_body,
        grid=(N // BR3, KS),
        in_specs=[
            pl.BlockSpec((BR3, CK), lambda i, k: (i, k)),
            pl.BlockSpec((N, D), lambda i, k: (0, 0)),
            pl.BlockSpec((D, D), lambda i, k: (0, 0)),
            pl.BlockSpec((1, D), lambda i, k: (0, 0)),
        ],
        out_specs=pl.BlockSpec((BR3, D), lambda i, k: (i, 0)),
        out_shape=jax.ShapeDtypeStruct((N, D), jnp.float32),
        scratch_shapes=[
            pltpu.VMEM((BR3, D), jnp.float32),
            pltpu.VMEM((BR3, 1), jnp.float32),
        ],
        compiler_params=pltpu.CompilerParams(
            dimension_semantics=("parallel", "arbitrary")),
    )(A, h1, W2, b2)


def kernel(z, W1, b1, W2, b2):
    xxc = jnp.sum(z ** 2, axis=1, keepdims=True)
    xxt = xxc.reshape(1, N)
    idx, h1 = _run_k1(z, xxt, xxc, W1, b1.reshape(1, D))
    a_flat = _run_k2(idx)
    A = a_flat.reshape(N, N)
    h = _run_k3(A, h1, W2, b2.reshape(1, D))
    return (h, A)
